# Initial kernel scaffold; baseline (speedup 1.0000x reference)
import jax
import jax.numpy as jnp
from jax.experimental import pallas as pl


def kernel(x, x_enc_mark, sp_embed, sp_W1, sp_b1, sp_W2, sp_b2, tm_embed, gru_Wih, gru_Whh, gru_bih, gru_bhh, gcn_W1, gcn_b1, gcn_W2, gcn_b2):
    raise NotImplementedError("write your pallas kernel here")


# speedup vs baseline: 2.1283x; 2.1283x over previous
import functools

import jax
import jax.numpy as jnp
from jax.experimental import pallas as pl
from jax.experimental.pallas import tpu as pltpu

_F32 = jnp.float32


def _elu(v):
    return jnp.where(v > 0, v, jnp.exp(jnp.minimum(v, 0.0)) - 1.0)


def _spatial_body(x_ref, se_ref, w1x_ref, w1s_ref, b1_ref, w2_ref, b2_ref,
                  o_ref):
    xb = x_ref[0]
    h = jax.lax.dot_general(xb, w1x_ref[...], (((0,), (0,)), ((), ())),
                            preferred_element_type=_F32)
    h = h + jnp.dot(se_ref[...], w1s_ref[...], preferred_element_type=_F32)
    h = _elu(h + b1_ref[...])
    o_ref[0] = jnp.dot(h, w2_ref[...], preferred_element_type=_F32) + b2_ref[...]


def _spatial_encoder(x, sp_embed, sp_W1, sp_b1, sp_W2, sp_b2):
    B, T, N = x.shape
    L = sp_W2.shape[1]
    BN = 512
    w1x = sp_W1[:T]
    w1s = sp_W1[T:]
    grid = (B, N // BN)
    return pl.pallas_call(
        _spatial_body,
        grid=grid,
        in_specs=[
            pl.BlockSpec((1, T, BN), lambda b, i: (b, 0, i)),
            pl.BlockSpec((BN, sp_embed.shape[1]), lambda b, i: (i, 0)),
            pl.BlockSpec(w1x.shape, lambda b, i: (0, 0)),
            pl.BlockSpec(w1s.shape, lambda b, i: (0, 0)),
            pl.BlockSpec((1, L), lambda b, i: (0, 0)),
            pl.BlockSpec(sp_W2.shape, lambda b, i: (0, 0)),
            pl.BlockSpec((1, L), lambda b, i: (0, 0)),
        ],
        out_specs=pl.BlockSpec((1, BN, L), lambda b, i: (b, i, 0)),
        out_shape=jax.ShapeDtypeStruct((B, N, L), _F32),
    )(x, sp_embed, w1x, w1s, sp_b1.reshape(1, L), sp_W2, sp_b2.reshape(1, L))


def _gru_body(xt_ref, wr_ref, wz_ref, wn_ref, whr_ref, whz_ref, whn_ref,
              bir_ref, biz_ref, bin_ref, bhr_ref, bhz_ref, bhn_ref,
              o_ref, gr_ref, gz_ref, gn_ref, *, T):
    xt = xt_ref[...]
    gr_ref[...] = jnp.dot(xt, wr_ref[...], preferred_element_type=_F32) + bir_ref[...]
    gz_ref[...] = jnp.dot(xt, wz_ref[...], preferred_element_type=_F32) + biz_ref[...]
    gn_ref[...] = jnp.dot(xt, wn_ref[...], preferred_element_type=_F32) + bin_ref[...]

    whr = whr_ref[...]
    whz = whz_ref[...]
    whn = whn_ref[...]
    bhr = bhr_ref[...]
    bhz = bhz_ref[...]
    bhn = bhn_ref[...]

    def step(t, h):
        sl = pl.ds(t * 8, 8)
        r = jax.nn.sigmoid(gr_ref[sl, :] +
                           jnp.dot(h, whr, preferred_element_type=_F32) + bhr)
        z = jax.nn.sigmoid(gz_ref[sl, :] +
                           jnp.dot(h, whz, preferred_element_type=_F32) + bhz)
        n = jnp.tanh(gn_ref[sl, :] +
                     r * (jnp.dot(h, whn, preferred_element_type=_F32) + bhn))
        hnew = (1.0 - z) * n + z * h
        o_ref[sl, :] = hnew
        return hnew

    jax.lax.fori_loop(0, T, step, jnp.zeros((8, 64), _F32), unroll=4)


def _temporal_encoder(x, x_enc_mark, tm_embed, gru_Wih, gru_Whh, gru_bih,
                      gru_bhh):
    B, T, N = x.shape
    L = gru_Whh.shape[1]
    gin = gru_Wih.shape[1]
    te = jnp.broadcast_to(tm_embed[None], (B, T, tm_embed.shape[1]))
    xt = jnp.concatenate([x, x_enc_mark, te], axis=2)
    xtt = jnp.transpose(xt, (1, 0, 2))
    xtt = jnp.concatenate(
        [xtt, jnp.zeros((T, 8 - B, gin), _F32)], axis=1)
    xt_flat = xtt.reshape(T * 8, gin)

    wr = gru_Wih[0 * L:1 * L].T
    wz = gru_Wih[1 * L:2 * L].T
    wn = gru_Wih[2 * L:3 * L].T
    whr = gru_Whh[0 * L:1 * L].T
    whz = gru_Whh[1 * L:2 * L].T
    whn = gru_Whh[2 * L:3 * L].T
    bir = gru_bih[0 * L:1 * L].reshape(1, L)
    biz = gru_bih[1 * L:2 * L].reshape(1, L)
    bin_ = gru_bih[2 * L:3 * L].reshape(1, L)
    bhr = gru_bhh[0 * L:1 * L].reshape(1, L)
    bhz = gru_bhh[1 * L:2 * L].reshape(1, L)
    bhn = gru_bhh[2 * L:3 * L].reshape(1, L)

    out = pl.pallas_call(
        functools.partial(_gru_body, T=T),
        out_shape=jax.ShapeDtypeStruct((T * 8, L), _F32),
        scratch_shapes=[
            pltpu.VMEM((T * 8, L), _F32),
            pltpu.VMEM((T * 8, L), _F32),
            pltpu.VMEM((T * 8, L), _F32),
        ],
    )(xt_flat, wr, wz, wn, whr, whz, whn, bir, biz, bin_, bhr, bhz, bhn)
    return jnp.transpose(out.reshape(T, 8, L)[:, :B], (1, 0, 2))


def _graph_body_l1(xi_ref, xf_ref, w_ref, b_ref, o_ref):
    xi = xi_ref[0]
    xf = xf_ref[0]
    s = jax.lax.dot_general(xi, xf, (((1,), (1,)), ((), ())),
                            preferred_element_type=_F32)
    a = jnp.where(s > 0, jnp.tanh(s), 0.0)
    deg = jnp.sum(a, axis=1, keepdims=True) + 1e-6
    p = jnp.dot(a, xf, preferred_element_type=_F32)
    h = jnp.dot(p / deg, w_ref[...], preferred_element_type=_F32) + b_ref[...]
    o_ref[0] = _elu(h)


def _graph_body_l2(xi_ref, xf_ref, f_ref, w_ref, b_ref, o_ref):
    xi = xi_ref[0]
    xf = xf_ref[0]
    s = jax.lax.dot_general(xi, xf, (((1,), (1,)), ((), ())),
                            preferred_element_type=_F32)
    a = jnp.where(s > 0, jnp.tanh(s), 0.0)
    deg = jnp.sum(a, axis=1, keepdims=True) + 1e-6
    p = jnp.dot(a, f_ref[0], preferred_element_type=_F32)
    o_ref[0] = jnp.dot(p / deg, w_ref[...],
                       preferred_element_type=_F32) + b_ref[...]


def _graph_layer1(X, W, b, BM):
    B, Mp, L = X.shape
    grid = (B, Mp // BM)
    return pl.pallas_call(
        _graph_body_l1,
        grid=grid,
        in_specs=[
            pl.BlockSpec((1, BM, L), lambda bb, i: (bb, i, 0)),
            pl.BlockSpec((1, Mp, L), lambda bb, i: (bb, 0, 0)),
            pl.BlockSpec((L, L), lambda bb, i: (0, 0)),
            pl.BlockSpec((1, L), lambda bb, i: (0, 0)),
        ],
        out_specs=pl.BlockSpec((1, BM, L), lambda bb, i: (bb, i, 0)),
        out_shape=jax.ShapeDtypeStruct((B, Mp, L), _F32),
    )(X, X, W, b.reshape(1, L))


def _graph_layer2(X, F, W, b, BM):
    B, Mp, L = X.shape
    grid = (B, Mp // BM)
    return pl.pallas_call(
        _graph_body_l2,
        grid=grid,
        in_specs=[
            pl.BlockSpec((1, BM, L), lambda bb, i: (bb, i, 0)),
            pl.BlockSpec((1, Mp, L), lambda bb, i: (bb, 0, 0)),
            pl.BlockSpec((1, Mp, L), lambda bb, i: (bb, 0, 0)),
            pl.BlockSpec((L, L), lambda bb, i: (0, 0)),
            pl.BlockSpec((1, L), lambda bb, i: (0, 0)),
        ],
        out_specs=pl.BlockSpec((1, BM, L), lambda bb, i: (bb, i, 0)),
        out_shape=jax.ShapeDtypeStruct((B, Mp, L), _F32),
    )(X, X, F, W, b.reshape(1, L))


def kernel(x, x_enc_mark, sp_embed, sp_W1, sp_b1, sp_W2, sp_b2, tm_embed,
           gru_Wih, gru_Whh, gru_bih, gru_bhh, gcn_W1, gcn_b1, gcn_W2,
           gcn_b2):
    B, T, N = x.shape
    L = gru_Whh.shape[1]
    M = N + T

    Xs = _spatial_encoder(x, sp_embed, sp_W1, sp_b1, sp_W2, sp_b2)
    Xt = _temporal_encoder(x, x_enc_mark, tm_embed, gru_Wih, gru_Whh,
                           gru_bih, gru_bhh)

    Mp = 2304
    X = jnp.concatenate(
        [Xs, Xt, jnp.zeros((B, Mp - M, L), _F32)], axis=1)

    BM = 384
    H1 = _graph_layer1(X, gcn_W1, gcn_b1, BM)
    H2 = _graph_layer2(X, H1, gcn_W2, gcn_b2, BM)
    return H2[:, :M]


# speedup vs baseline: 2.1547x; 1.0124x over previous
import jax
import jax.numpy as jnp
from jax.experimental import pallas as pl
from jax.experimental.pallas import tpu as pltpu

_F32 = jnp.float32

_B = 4
_T = 168
_N = 2048
_L = 64
_M = _N + _T
_MP = 2304
_BM = 384


def _elu(v):
    return jnp.where(v > 0, v, jnp.exp(jnp.minimum(v, 0.0)) - 1.0)


def _encoder_body(x_ref, marks_ref, se_ref, w1x_ref, w1s_ref, b1_ref, w2_ref,
                  b2_ref, tm_ref, wih_x_ref, wih_m_ref, wih_t_ref, bihc_ref,
                  whhc_ref, bhhc_ref, o_ref, g0_sc, g1_sc, g2_sc, g3_sc):
    seproj = (jnp.dot(se_ref[...], w1s_ref[...], preferred_element_type=_F32)
              + b1_ref[...])
    w2 = w2_ref[...]
    b2 = b2_ref[...]
    for b in range(_B):
        xb = x_ref[b]
        h = jax.lax.dot_general(xb, w1x_ref[...], (((0,), (0,)), ((), ())),
                                preferred_element_type=_F32)
        h = _elu(h + seproj)
        o_ref[b * _MP:b * _MP + _N, :] = (
            jnp.dot(h, w2, preferred_element_type=_F32) + b2)
        o_ref[b * _MP + _M:(b + 1) * _MP, :] = jnp.zeros((_MP - _M, _L), _F32)

    teproj = (jnp.dot(tm_ref[...], wih_t_ref[...], preferred_element_type=_F32)
              + bihc_ref[...])
    gi_refs = (g0_sc, g1_sc, g2_sc, g3_sc)
    for b in range(_B):
        gi = jnp.dot(x_ref[b], wih_x_ref[...], preferred_element_type=_F32)
        gi = gi + jnp.dot(marks_ref[b], wih_m_ref[...],
                          preferred_element_type=_F32)
        gi_refs[b][...] = gi + teproj

    whhc = whhc_ref[...]
    bhhc = bhhc_ref[...]

    def step(t, h):
        gi_t = jnp.concatenate(
            [gi_refs[b][pl.ds(t, 1), :] for b in range(_B)], axis=0)
        gh = jnp.dot(h, whhc, preferred_element_type=_F32) + bhhc
        r = jax.nn.sigmoid(gi_t[:, 0 * _L:1 * _L] + gh[:, 0 * _L:1 * _L])
        z = jax.nn.sigmoid(gi_t[:, 1 * _L:2 * _L] + gh[:, 1 * _L:2 * _L])
        n = jnp.tanh(gi_t[:, 2 * _L:3 * _L] + r * gh[:, 2 * _L:3 * _L])
        hnew = (1.0 - z) * n + z * h
        for b in range(_B):
            o_ref[pl.ds(b * _MP + _N + t, 1), :] = hnew[b:b + 1]
        return hnew

    jax.lax.fori_loop(0, _T, step, jnp.zeros((_B, _L), _F32))


def _graph_body_l1(xi_ref, xf_ref, w_ref, b_ref, o_ref):
    xi = xi_ref[0]
    xf = xf_ref[0]
    s = jax.lax.dot_general(xi, xf, (((1,), (1,)), ((), ())),
                            preferred_element_type=_F32)
    a = jnp.where(s > 0, jnp.tanh(s), 0.0)
    deg = jnp.sum(a, axis=1, keepdims=True) + 1e-6
    p = jnp.dot(a, xf, preferred_element_type=_F32)
    h = jnp.dot(p / deg, w_ref[...], preferred_element_type=_F32) + b_ref[...]
    o_ref[0] = _elu(h)


def _graph_body_l2(xi_ref, xf_ref, f_ref, w_ref, b_ref, o_ref):
    xi = xi_ref[0]
    xf = xf_ref[0]
    s = jax.lax.dot_general(xi, xf, (((1,), (1,)), ((), ())),
                            preferred_element_type=_F32)
    a = jnp.where(s > 0, jnp.tanh(s), 0.0)
    deg = jnp.sum(a, axis=1, keepdims=True) + 1e-6
    p = jnp.dot(a, f_ref[0], preferred_element_type=_F32)
    o_ref[0] = jnp.dot(p / deg, w_ref[...],
                       preferred_element_type=_F32) + b_ref[...]


def kernel(x, x_enc_mark, sp_embed, sp_W1, sp_b1, sp_W2, sp_b2, tm_embed,
           gru_Wih, gru_Whh, gru_bih, gru_bhh, gcn_W1, gcn_b1, gcn_W2,
           gcn_b2):
    L = _L
    w1x = sp_W1[:_T]
    w1s = sp_W1[_T:]
    wih_t_full = gru_Wih.T
    wih_x = wih_t_full[:_N]
    wih_m = wih_t_full[_N:_N + 4]
    wih_tm = wih_t_full[_N + 4:]
    whhc = gru_Whh.T

    x_flat = pl.pallas_call(
        _encoder_body,
        out_shape=jax.ShapeDtypeStruct((_B * _MP, L), _F32),
        scratch_shapes=[
            pltpu.VMEM((_T, 3 * L), _F32),
            pltpu.VMEM((_T, 3 * L), _F32),
            pltpu.VMEM((_T, 3 * L), _F32),
            pltpu.VMEM((_T, 3 * L), _F32),
        ],
    )(x, x_enc_mark, sp_embed, w1x, w1s, sp_b1.reshape(1, L), sp_W2,
      sp_b2.reshape(1, L), tm_embed, wih_x, wih_m, wih_tm,
      gru_bih.reshape(1, 3 * L), whhc, gru_bhh.reshape(1, 3 * L))

    X = x_flat.reshape(_B, _MP, L)

    grid = (_B, _MP // _BM)
    H1 = pl.pallas_call(
        _graph_body_l1,
        grid=grid,
        in_specs=[
            pl.BlockSpec((1, _BM, L), lambda bb, i: (bb, i, 0)),
            pl.BlockSpec((1, _MP, L), lambda bb, i: (bb, 0, 0)),
            pl.BlockSpec((L, L), lambda bb, i: (0, 0)),
            pl.BlockSpec((1, L), lambda bb, i: (0, 0)),
        ],
        out_specs=pl.BlockSpec((1, _BM, L), lambda bb, i: (bb, i, 0)),
        out_shape=jax.ShapeDtypeStruct((_B, _MP, L), _F32),
    )(X, X, gcn_W1, gcn_b1.reshape(1, L))

    H2 = pl.pallas_call(
        _graph_body_l2,
        grid=grid,
        in_specs=[
            pl.BlockSpec((1, _BM, L), lambda bb, i: (bb, i, 0)),
            pl.BlockSpec((1, _MP, L), lambda bb, i: (bb, 0, 0)),
            pl.BlockSpec((1, _MP, L), lambda bb, i: (bb, 0, 0)),
            pl.BlockSpec((L, L), lambda bb, i: (0, 0)),
            pl.BlockSpec((1, L), lambda bb, i: (0, 0)),
        ],
        out_specs=pl.BlockSpec((1, _BM, L), lambda bb, i: (bb, i, 0)),
        out_shape=jax.ShapeDtypeStruct((_B, _MP, L), _F32),
    )(X, X, H1, gcn_W2, gcn_b2.reshape(1, L))
    return H2[:, :_M]


# speedup vs baseline: 2.7619x; 1.2818x over previous
import jax
import jax.numpy as jnp
from jax.experimental import pallas as pl
from jax.experimental.pallas import tpu as pltpu

_F32 = jnp.float32
_BF16 = jnp.bfloat16

_B = 4
_T = 168
_N = 2048
_L = 64
_M = _N + _T
_MP = 2304
_BM = 576


def _elu(v):
    return jnp.where(v > 0, v, jnp.exp(jnp.minimum(v, 0.0)) - 1.0)


def _sigmoid(v):
    return 0.5 * jnp.tanh(0.5 * v) + 0.5


def _encoder_body(x_ref, marks_ref, se_ref, w1x_ref, w1s_ref, b1_ref, w2_ref,
                  b2_ref, tm_ref, wxr_ref, wxz_ref, wxn_ref, wmr_ref,
                  wmz_ref, wmn_ref, ter_ref, tez_ref, ten_ref,
                  whr_ref, whz_ref, whn_ref, bhn_ref, o_ref, *gi_sc):
    seproj = (jnp.dot(se_ref[...], w1s_ref[...], preferred_element_type=_F32)
              + b1_ref[...])
    w2 = w2_ref[...]
    b2 = b2_ref[...]
    for b in range(_B):
        xb = x_ref[b]
        h = jax.lax.dot_general(xb, w1x_ref[...], (((0,), (0,)), ((), ())),
                                preferred_element_type=_F32)
        h = _elu(h + seproj)
        o_ref[b * _MP:b * _MP + _N, :] = (
            jnp.dot(h, w2, preferred_element_type=_F32) + b2)
        o_ref[b * _MP + _M:(b + 1) * _MP, :] = jnp.zeros((_MP - _M, _L), _F32)

    wx = (wxr_ref, wxz_ref, wxn_ref)
    wm = (wmr_ref, wmz_ref, wmn_ref)
    te = (ter_ref, tez_ref, ten_ref)
    for g in range(3):
        teproj = jnp.dot(tm_ref[...], te[g][...],
                         preferred_element_type=_F32)
        for b in range(_B):
            gi = jnp.dot(x_ref[b], wx[g][...], preferred_element_type=_F32)
            gi = gi + jnp.dot(marks_ref[b], wm[g][...],
                              preferred_element_type=_F32)
            gi_sc[g * _B + b][...] = gi + teproj

    whr = whr_ref[...]
    whz = whz_ref[...]
    whn = whn_ref[...]
    bhn = bhn_ref[...]

    def step(t, h):
        gir = jnp.concatenate(
            [gi_sc[0 * _B + b][pl.ds(t, 1), :] for b in range(_B)], axis=0)
        giz = jnp.concatenate(
            [gi_sc[1 * _B + b][pl.ds(t, 1), :] for b in range(_B)], axis=0)
        gin = jnp.concatenate(
            [gi_sc[2 * _B + b][pl.ds(t, 1), :] for b in range(_B)], axis=0)
        ghr = jnp.dot(h, whr, preferred_element_type=_F32)
        ghz = jnp.dot(h, whz, preferred_element_type=_F32)
        ghn = jnp.dot(h, whn, preferred_element_type=_F32)
        r = _sigmoid(gir + ghr)
        z = _sigmoid(giz + ghz)
        n = jnp.tanh(gin + r * (ghn + bhn))
        hnew = n + z * (h - n)
        for b in range(_B):
            o_ref[pl.ds(b * _MP + _N + t, 1), :] = hnew[b:b + 1]
        return hnew

    jax.lax.fori_loop(0, _T, step, jnp.zeros((_B, _L), _F32), unroll=4)


def _graph_body_l1(xi_ref, xf_ref, w_ref, b_ref, o_ref):
    xi = xi_ref[0].astype(_BF16)
    xff = xf_ref[0]
    xf = xff.astype(_BF16)
    s = jax.lax.dot_general(xi, xf, (((1,), (1,)), ((), ())),
                            preferred_element_type=_F32)
    a = jnp.tanh(jnp.maximum(s, 0.0))
    deg = jnp.sum(a, axis=1, keepdims=True) + 1e-6
    p = jnp.dot(a.astype(_BF16), xf, preferred_element_type=_F32)
    h = jnp.dot(p / deg, w_ref[...], preferred_element_type=_F32) + b_ref[...]
    o_ref[0] = _elu(h)


def _graph_body_l2(xi_ref, xf_ref, f_ref, w_ref, b_ref, o_ref):
    xi = xi_ref[0].astype(_BF16)
    xf = xf_ref[0].astype(_BF16)
    s = jax.lax.dot_general(xi, xf, (((1,), (1,)), ((), ())),
                            preferred_element_type=_F32)
    a = jnp.tanh(jnp.maximum(s, 0.0))
    deg = jnp.sum(a, axis=1, keepdims=True) + 1e-6
    p = jnp.dot(a.astype(_BF16), f_ref[0].astype(_BF16),
                preferred_element_type=_F32)
    o_ref[0] = jnp.dot(p / deg, w_ref[...],
                       preferred_element_type=_F32) + b_ref[...]


def kernel(x, x_enc_mark, sp_embed, sp_W1, sp_b1, sp_W2, sp_b2, tm_embed,
           gru_Wih, gru_Whh, gru_bih, gru_bhh, gcn_W1, gcn_b1, gcn_W2,
           gcn_b2):
    L = _L
    w1x = sp_W1[:_T]
    w1s = sp_W1[_T:]
    wih_t_full = gru_Wih.T
    whh_t = gru_Whh.T
    wxg = [wih_t_full[:_N, g * L:(g + 1) * L] for g in range(3)]
    wmg = [wih_t_full[_N:_N + 4, g * L:(g + 1) * L] for g in range(3)]
    wtg = [wih_t_full[_N + 4:, g * L:(g + 1) * L] for g in range(3)]
    whg = [whh_t[:, g * L:(g + 1) * L] for g in range(3)]
    tm_aug = jnp.concatenate(
        [tm_embed, jnp.ones((_T, 1), _F32)], axis=1)
    bihg = [gru_bih[g * L:(g + 1) * L] for g in range(3)]
    bhhg = [gru_bhh[g * L:(g + 1) * L] for g in range(3)]
    ter = jnp.concatenate([wtg[0], (bihg[0] + bhhg[0]).reshape(1, L)], axis=0)
    tez = jnp.concatenate([wtg[1], (bihg[1] + bhhg[1]).reshape(1, L)], axis=0)
    ten = jnp.concatenate([wtg[2], bihg[2].reshape(1, L)], axis=0)

    x_flat = pl.pallas_call(
        _encoder_body,
        out_shape=jax.ShapeDtypeStruct((_B * _MP, L), _F32),
        scratch_shapes=[pltpu.VMEM((_T, L), _F32)] * 12,
    )(x, x_enc_mark, sp_embed, w1x, w1s, sp_b1.reshape(1, L), sp_W2,
      sp_b2.reshape(1, L), tm_aug, wxg[0], wxg[1], wxg[2],
      wmg[0], wmg[1], wmg[2], ter, tez, ten,
      whg[0], whg[1], whg[2], bhhg[2].reshape(1, L))

    X = x_flat.reshape(_B, _MP, L)

    grid = (_B, _MP // _BM)
    H1 = pl.pallas_call(
        _graph_body_l1,
        grid=grid,
        in_specs=[
            pl.BlockSpec((1, _BM, L), lambda bb, i: (bb, i, 0)),
            pl.BlockSpec((1, _MP, L), lambda bb, i: (bb, 0, 0)),
            pl.BlockSpec((L, L), lambda bb, i: (0, 0)),
            pl.BlockSpec((1, L), lambda bb, i: (0, 0)),
        ],
        out_specs=pl.BlockSpec((1, _BM, L), lambda bb, i: (bb, i, 0)),
        out_shape=jax.ShapeDtypeStruct((_B, _MP, L), _F32),
    )(X, X, gcn_W1, gcn_b1.reshape(1, L))

    H2 = pl.pallas_call(
        _graph_body_l2,
        grid=grid,
        in_specs=[
            pl.BlockSpec((1, _BM, L), lambda bb, i: (bb, i, 0)),
            pl.BlockSpec((1, _MP, L), lambda bb, i: (bb, 0, 0)),
            pl.BlockSpec((1, _MP, L), lambda bb, i: (bb, 0, 0)),
            pl.BlockSpec((L, L), lambda bb, i: (0, 0)),
            pl.BlockSpec((1, L), lambda bb, i: (0, 0)),
        ],
        out_specs=pl.BlockSpec((1, _BM, L), lambda bb, i: (bb, i, 0)),
        out_shape=jax.ShapeDtypeStruct((_B, _M, L), _F32),
    )(X, X, H1, gcn_W2, gcn_b2.reshape(1, L))
    return H2
